# baseline (speedup 1.0000x reference)
import functools

import jax
import jax.numpy as jnp
from jax import lax
from jax.experimental import pallas as pl
from jax.experimental.pallas import tpu as pltpu
from jax.experimental.pallas import tpu_sc as plsc

N = 100000
E = 1600000
G = 64
H = 32
NC, NS, L = 2, 16, 16
NW = NC * NS
NP = 102400
PT = NP // NS
RND = NP // 128
EW = E // NW
CD = 10000
NCHD = EW // CD
C = 2000
NCH = EW // C

_mesh = plsc.VectorSubcoreMesh(core_axis_name="c", subcore_axis_name="s")


@functools.partial(
    pl.kernel,
    out_type=jax.ShapeDtypeStruct((NC * NP,), jnp.float32),
    mesh=_mesh,
    scratch_types=[
        [pltpu.VMEM((CD,), jnp.int32)] * 3,
        [pltpu.VMEM((CD,), jnp.float32)] * 3,
        pltpu.VMEM((PT,), jnp.float32),
        pltpu.VMEM_SHARED((NP,), jnp.float32),
        [pltpu.SemaphoreType.DMA] * 3,
        [pltpu.SemaphoreType.DMA] * 3,
        [pltpu.SemaphoreType.DMA] * 3,
    ],
    compiler_params=pltpu.CompilerParams(needs_layout_passes=False),
)
def _deg_kernel(dst_hbm, ew_hbm, out_hbm, idx_v, val_v, zbuf, acc,
                lsems_i, lsems_v, ssems):
    c = lax.axis_index("c")
    s = lax.axis_index("s")
    wid = s * NC + c

    def zb(i, _):
        zbuf[pl.ds(i * L, L)] = jnp.zeros((L,), jnp.float32)
        return 0
    lax.fori_loop(0, PT // L, zb, 0)
    pltpu.sync_copy(zbuf, acc.at[pl.ds(s * PT, PT)])
    plsc.subcore_barrier()
    base = wid * EW

    def start_loads(j):
        b = j % 3
        off = base + j * CD
        di = pltpu.async_copy(dst_hbm.at[pl.ds(off, CD)], idx_v[b],
                              lsems_i[b])
        dv = pltpu.async_copy(ew_hbm.at[pl.ds(off, CD)], val_v[b],
                              lsems_v[b])
        return di, dv

    loads = {0: start_loads(0)}
    scats = {}
    for j in range(NCHD):
        b = j % 3
        di, dv = loads.pop(j)
        di.wait()
        dv.wait()
        if j + 1 < NCHD:
            if j - 2 >= 0:
                scats.pop(j - 2).wait()
            loads[j + 1] = start_loads(j + 1)
        scats[j] = pltpu.async_copy(val_v[b], acc.at[idx_v[b]],
                                    ssems[b], add=True)
    for j in sorted(scats):
        scats.pop(j).wait()
    plsc.subcore_barrier()
    pltpu.sync_copy(acc.at[pl.ds(s * PT, PT)],
                    out_hbm.at[pl.ds(c * NP + s * PT, PT)])


@functools.partial(
    pl.kernel,
    out_type=jax.ShapeDtypeStruct((NC * NP,), jnp.float32),
    mesh=_mesh,
    scratch_types=[
        pltpu.VMEM((NP,), jnp.float32),
        [pltpu.VMEM((C,), jnp.int32)] * 2,
        [pltpu.VMEM((C,), jnp.int32)] * 3,
        [pltpu.VMEM((C,), jnp.float32)] * 2,
        [pltpu.VMEM((C,), jnp.float32)] * 3,
        pltpu.VMEM_SHARED((NP,), jnp.float32),
        [pltpu.SemaphoreType.DMA] * 2,
        [pltpu.SemaphoreType.DMA] * 3,
        [pltpu.SemaphoreType.DMA] * 2,
        [pltpu.SemaphoreType.DMA] * 3,
        pltpu.SemaphoreType.DMA,
    ],
    compiler_params=pltpu.CompilerParams(needs_layout_passes=False),
)
def _edge_kernel(src_hbm, dst_hbm, ew_hbm, tbl_hbm, out_hbm,
                 tbl_v, sidx, didx, w_v, prod, acc,
                 sem_s, sem_d, sem_w, sem_sc, sem_t):
    c = lax.axis_index("c")
    s = lax.axis_index("s")
    wid = s * NC + c
    tload = pltpu.async_copy(tbl_hbm, tbl_v, sem_t)

    def zb(i, _):
        prod[0][pl.ds(i * L, L)] = jnp.zeros((L,), jnp.float32)
        return 0
    lax.fori_loop(0, C // L, zb, 0)
    for r in range(PT // C):
        pltpu.sync_copy(prod[0], acc.at[pl.ds(s * PT + r * C, C)])
    rem = PT % C
    if rem:
        pltpu.sync_copy(prod[0].at[pl.ds(0, rem)],
                        acc.at[pl.ds(s * PT + (PT // C) * C, rem)])
    tload.wait()
    plsc.subcore_barrier()
    base = wid * EW

    def start_loads(j):
        b2, b3 = j % 2, j % 3
        off = base + j * C
        ds_ = pltpu.async_copy(src_hbm.at[pl.ds(off, C)], sidx[b2],
                               sem_s[b2])
        dd = pltpu.async_copy(dst_hbm.at[pl.ds(off, C)], didx[b3],
                              sem_d[b3])
        dw = pltpu.async_copy(ew_hbm.at[pl.ds(off, C)], w_v[b2],
                              sem_w[b2])
        return ds_, dd, dw

    loads = {0: start_loads(0)}
    scats = {}
    for j in range(NCH):
        b2, b3 = j % 2, j % 3
        for d in loads.pop(j):
            d.wait()
        if j + 1 < NCH:
            if j - 2 >= 0:
                scats.pop(j - 2).wait()
            loads[j + 1] = start_loads(j + 1)

        def inner(i, _, b2=b2, b3=b3):
            for u in range(5):
                o = i * (5 * L) + u * L
                s16 = sidx[b2][pl.ds(o, L)]
                g16 = plsc.load_gather(tbl_v, [s16])
                prod[b3][pl.ds(o, L)] = g16 * w_v[b2][pl.ds(o, L)]
            return 0

        lax.fori_loop(0, C // (5 * L), inner, 0)
        scats[j] = pltpu.async_copy(prod[b3], acc.at[didx[b3]],
                                    sem_sc[b3], add=True)
    for j in sorted(scats):
        scats.pop(j).wait()
    plsc.subcore_barrier()
    pltpu.sync_copy(acc.at[pl.ds(s * PT, PT)],
                    out_hbm.at[pl.ds(c * NP + s * PT, PT)])


def _dinv_body(degp_ref, x_ref, dinv_ref, p_ref):
    deg = degp_ref[0] + degp_ref[1] + 1.0
    dinv = lax.rsqrt(deg)
    dinv_ref[...] = dinv
    p_ref[...] = dinv * x_ref[...]


_dinv_call = pl.pallas_call(
    _dinv_body,
    out_shape=(jax.ShapeDtypeStruct((RND, 128), jnp.float32),
               jax.ShapeDtypeStruct((RND, 128), jnp.float32)),
)


def _mlp_body(s1p_ref, dinv_ref, p_ref, w1_ref, b1_ref, w2_ref, q_ref):
    dinv = dinv_ref[...]
    a = dinv * (s1p_ref[0] + s1p_ref[1] + p_ref[...])
    t = jnp.zeros_like(a)
    for h in range(H):
        t = t + jnp.maximum(a * w1_ref[0, h] + b1_ref[0, h], 0.0) * w2_ref[0, h]
    q_ref[...] = dinv * t


_mlp_call = pl.pallas_call(
    _mlp_body,
    out_shape=jax.ShapeDtypeStruct((RND, 128), jnp.float32),
)


def _final_body(s2p_ref, dinv_ref, q_ref, batch_ref, b2_ref, out_ref):
    cv = dinv_ref[...] * (s2p_ref[0] + s2p_ref[1] + q_ref[...]) + b2_ref[0, 0]
    b = batch_ref[...]
    sums, cnts = [], []
    for g in range(G):
        m = b == g
        sums.append(jnp.sum(jnp.where(m, cv, 0.0)))
        cnts.append(jnp.sum(jnp.where(m, 1.0, 0.0)))
    out_ref[0, :] = jnp.stack(sums) / jnp.maximum(jnp.stack(cnts), 1.0)


_final_call = pl.pallas_call(
    _final_body,
    out_shape=jax.ShapeDtypeStruct((1, G), jnp.float32),
)


def kernel(x, edge_index, edge_weight, batch, W1, b1, W2, b2):
    src = edge_index[0]
    dst = edge_index[1]
    x_p = jnp.pad(x, (0, NP - N)).reshape(RND, 128)
    batch_p = jnp.pad(batch, (0, NP - N), constant_values=G).reshape(RND, 128)
    w1 = W1.reshape(1, H)
    b1r = b1.reshape(1, H)
    w2 = W2.reshape(1, H)
    b2r = b2.reshape(1, 1)

    degp = _deg_kernel(dst, edge_weight).reshape(NC, RND, 128)
    dinv2, p2 = _dinv_call(degp, x_p)
    s1p = _edge_kernel(src, dst, edge_weight,
                       p2.reshape(NP)).reshape(NC, RND, 128)
    q2 = _mlp_call(s1p, dinv2, p2, w1, b1r, w2)
    s2p = _edge_kernel(src, dst, edge_weight,
                       q2.reshape(NP)).reshape(NC, RND, 128)
    out2 = _final_call(s2p, dinv2, q2, batch_p, b2r)
    return out2.reshape(G)


# speedup vs baseline: 1.0179x; 1.0179x over previous
import functools

import jax
import jax.numpy as jnp
from jax import lax
from jax.experimental import pallas as pl
from jax.experimental.pallas import tpu as pltpu
from jax.experimental.pallas import tpu_sc as plsc

N = 100000
E = 1600000
G = 64
H = 32
NC, NS, L = 2, 16, 16
NW = NC * NS
NP = 102400
PT = NP // NS
RND = NP // 128

CD = 6400
NCHD = E // CD
RD = NCHD // NW
TAILD = NCHD - RD * NW

C = 1280
NCHT = E // C
RE = NCHT // NW
TAILE = NCHT - RE * NW

_mesh = plsc.VectorSubcoreMesh(core_axis_name="c", subcore_axis_name="s")


@functools.partial(
    pl.kernel,
    out_type=jax.ShapeDtypeStruct((NC * NP,), jnp.float32),
    mesh=_mesh,
    scratch_types=[
        [pltpu.VMEM((2, CD), jnp.int32)] * 3,
        [pltpu.VMEM((CD,), jnp.int32)] * 3,
        [pltpu.VMEM((CD,), jnp.float32)] * 3,
        pltpu.VMEM((PT,), jnp.float32),
        pltpu.VMEM_SHARED((NP,), jnp.float32),
        [pltpu.SemaphoreType.DMA] * 3,
        [pltpu.SemaphoreType.DMA] * 3,
        [pltpu.SemaphoreType.DMA] * 3,
    ],
    compiler_params=pltpu.CompilerParams(needs_layout_passes=False),
)
def _deg_kernel(ei_hbm, ew_hbm, out_hbm, idx_v, didx, val_v, zbuf, acc,
                lsems_i, lsems_v, ssems):
    c = lax.axis_index("c")
    s = lax.axis_index("s")
    wid = s * NC + c

    def zb(i, _):
        zbuf[pl.ds(i * L, L)] = jnp.zeros((L,), jnp.float32)
        return 0
    lax.fori_loop(0, PT // L, zb, 0)
    pltpu.sync_copy(zbuf, acc.at[pl.ds(s * PT, PT)])
    plsc.subcore_barrier()

    def start_loads(j):
        b = j % 3
        off = (j * NW + wid) * CD
        di = pltpu.async_copy(ei_hbm.at[:, pl.ds(off, CD)], idx_v[b],
                              lsems_i[b])
        dv = pltpu.async_copy(ew_hbm.at[pl.ds(off, CD)], val_v[b],
                              lsems_v[b])
        return di, dv

    loads = {0: start_loads(0)}
    scats = {}
    for j in range(RD):
        b = j % 3
        di, dv = loads.pop(j)
        di.wait()
        dv.wait()
        if j + 1 < RD:
            if j - 2 >= 0:
                scats.pop(j - 2).wait()
            loads[j + 1] = start_loads(j + 1)
        def dcopy(i, _, b=b):
            for u in range(4):
                o = i * (4 * L) + u * L
                didx[b][pl.ds(o, L)] = idx_v[b][1, pl.ds(o, L)]
            return 0
        lax.fori_loop(0, CD // (4 * L), dcopy, 0)
        scats[j] = pltpu.async_copy(val_v[b], acc.at[didx[b]],
                                    ssems[b], add=True)
    for j in sorted(scats):
        scats.pop(j).wait()

    @pl.when(wid < TAILD)
    def _tail():
        off = (RD * NW + wid) * CD
        pltpu.sync_copy(ei_hbm.at[:, pl.ds(off, CD)], idx_v[0])
        pltpu.sync_copy(ew_hbm.at[pl.ds(off, CD)], val_v[0])
        def dcopy_t(i, _):
            for u in range(4):
                o = i * (4 * L) + u * L
                didx[0][pl.ds(o, L)] = idx_v[0][1, pl.ds(o, L)]
            return 0
        lax.fori_loop(0, CD // (4 * L), dcopy_t, 0)
        pltpu.sync_copy(val_v[0], acc.at[didx[0]], add=True)

    plsc.subcore_barrier()
    pltpu.sync_copy(acc.at[pl.ds(s * PT, PT)],
                    out_hbm.at[pl.ds(c * NP + s * PT, PT)])


@functools.partial(
    pl.kernel,
    out_type=jax.ShapeDtypeStruct((NC * NP,), jnp.float32),
    mesh=_mesh,
    scratch_types=[
        pltpu.VMEM((NP,), jnp.float32),
        [pltpu.VMEM((2, C), jnp.int32)] * 3,
        [pltpu.VMEM((C,), jnp.int32)] * 3,
        [pltpu.VMEM((C,), jnp.float32)] * 2,
        [pltpu.VMEM((C,), jnp.float32)] * 3,
        pltpu.VMEM_SHARED((NP,), jnp.float32),
        [pltpu.SemaphoreType.DMA] * 3,
        [pltpu.SemaphoreType.DMA] * 2,
        [pltpu.SemaphoreType.DMA] * 3,
        pltpu.SemaphoreType.DMA,
    ],
    compiler_params=pltpu.CompilerParams(needs_layout_passes=False),
)
def _edge_kernel(ei_hbm, ew_hbm, tbl_hbm, out_hbm,
                 tbl_v, exy, didx, w_v, prod, acc,
                 sem_e, sem_w, sem_sc, sem_t):
    c = lax.axis_index("c")
    s = lax.axis_index("s")
    wid = s * NC + c
    tload = pltpu.async_copy(tbl_hbm, tbl_v, sem_t)

    def zb(i, _):
        prod[0][pl.ds(i * L, L)] = jnp.zeros((L,), jnp.float32)
        return 0
    lax.fori_loop(0, C // L, zb, 0)
    for r in range(PT // C):
        pltpu.sync_copy(prod[0], acc.at[pl.ds(s * PT + r * C, C)])
    tload.wait()
    plsc.subcore_barrier()

    def start_loads(j):
        b3, b2 = j % 3, j % 2
        off = (j * NW + wid) * C
        de = pltpu.async_copy(ei_hbm.at[:, pl.ds(off, C)], exy[b3],
                              sem_e[b3])
        dw = pltpu.async_copy(ew_hbm.at[pl.ds(off, C)], w_v[b2],
                              sem_w[b2])
        return de, dw

    def compute(b3, b2):
        def inner(i, _):
            for u in range(5):
                o = i * (5 * L) + u * L
                s16 = exy[b3][0, pl.ds(o, L)]
                g16 = plsc.load_gather(tbl_v, [s16])
                prod[b3][pl.ds(o, L)] = g16 * w_v[b2][pl.ds(o, L)]
                didx[b3][pl.ds(o, L)] = exy[b3][1, pl.ds(o, L)]
            return 0
        lax.fori_loop(0, C // (5 * L), inner, 0)

    loads = {0: start_loads(0)}
    scats = {}
    for j in range(RE):
        b3, b2 = j % 3, j % 2
        for d in loads.pop(j):
            d.wait()
        if j + 1 < RE:
            if j - 2 >= 0:
                scats.pop(j - 2).wait()
            loads[j + 1] = start_loads(j + 1)
        compute(b3, b2)
        scats[j] = pltpu.async_copy(prod[b3], acc.at[didx[b3]],
                                    sem_sc[b3], add=True)
    for j in sorted(scats):
        scats.pop(j).wait()

    @pl.when(wid < TAILE)
    def _tail():
        off = (RE * NW + wid) * C
        pltpu.sync_copy(ei_hbm.at[:, pl.ds(off, C)], exy[0])
        pltpu.sync_copy(ew_hbm.at[pl.ds(off, C)], w_v[0])
        compute(0, 0)
        pltpu.sync_copy(prod[0], acc.at[didx[0]], add=True)

    plsc.subcore_barrier()
    pltpu.sync_copy(acc.at[pl.ds(s * PT, PT)],
                    out_hbm.at[pl.ds(c * NP + s * PT, PT)])


def _dinv_body(degp_ref, x_ref, dinv_ref, p_ref):
    deg = degp_ref[0] + degp_ref[1] + 1.0
    dinv = lax.rsqrt(deg)
    dinv_ref[...] = dinv
    p_ref[...] = dinv * x_ref[...]


_dinv_call = pl.pallas_call(
    _dinv_body,
    out_shape=(jax.ShapeDtypeStruct((RND, 128), jnp.float32),
               jax.ShapeDtypeStruct((RND, 128), jnp.float32)),
)


def _mlp_body(s1p_ref, dinv_ref, p_ref, w1_ref, b1_ref, w2_ref, q_ref):
    dinv = dinv_ref[...]
    a = dinv * (s1p_ref[0] + s1p_ref[1] + p_ref[...])
    t = jnp.zeros_like(a)
    for h in range(H):
        t = t + jnp.maximum(a * w1_ref[0, h] + b1_ref[0, h], 0.0) * w2_ref[0, h]
    q_ref[...] = dinv * t


_mlp_call = pl.pallas_call(
    _mlp_body,
    out_shape=jax.ShapeDtypeStruct((RND, 128), jnp.float32),
)


def _final_body(s2p_ref, dinv_ref, q_ref, batch_ref, b2_ref, out_ref):
    cv = dinv_ref[...] * (s2p_ref[0] + s2p_ref[1] + q_ref[...]) + b2_ref[0, 0]
    b = batch_ref[...]
    sums, cnts = [], []
    for g in range(G):
        m = b == g
        sums.append(jnp.sum(jnp.where(m, cv, 0.0)))
        cnts.append(jnp.sum(jnp.where(m, 1.0, 0.0)))
    out_ref[0, :] = jnp.stack(sums) / jnp.maximum(jnp.stack(cnts), 1.0)


_final_call = pl.pallas_call(
    _final_body,
    out_shape=jax.ShapeDtypeStruct((1, G), jnp.float32),
)


def kernel(x, edge_index, edge_weight, batch, W1, b1, W2, b2):
    x_p = jnp.pad(x, (0, NP - N)).reshape(RND, 128)
    batch_p = jnp.pad(batch, (0, NP - N), constant_values=G).reshape(RND, 128)
    w1 = W1.reshape(1, H)
    b1r = b1.reshape(1, H)
    w2 = W2.reshape(1, H)
    b2r = b2.reshape(1, 1)

    degp = _deg_kernel(edge_index, edge_weight).reshape(NC, RND, 128)
    dinv2, p2 = _dinv_call(degp, x_p)
    s1p = _edge_kernel(edge_index, edge_weight,
                       p2.reshape(NP)).reshape(NC, RND, 128)
    q2 = _mlp_call(s1p, dinv2, p2, w1, b1r, w2)
    s2p = _edge_kernel(edge_index, edge_weight,
                       q2.reshape(NP)).reshape(NC, RND, 128)
    out2 = _final_call(s2p, dinv2, q2, batch_p, b2r)
    return out2.reshape(G)


# speedup vs baseline: 1.3299x; 1.3064x over previous
import functools

import jax
import jax.numpy as jnp
from jax import lax
from jax.experimental import pallas as pl
from jax.experimental.pallas import tpu as pltpu
from jax.experimental.pallas import tpu_sc as plsc

N = 100000
E = 1600000
G = 64
H = 32
NC, NS, L = 2, 16, 16
NW = NC * NS
NP = 102400
PT = NP // NS
RND = NP // 128

CD = 6400
NCHD = E // CD
RDN = NCHD // NW
TAILD = NCHD - RDN * NW

EW = E // NW
C = 2000
NCH = EW // C

_mesh = plsc.VectorSubcoreMesh(core_axis_name="c", subcore_axis_name="s")


@functools.partial(
    pl.kernel,
    out_type=(jax.ShapeDtypeStruct((NC * NP,), jnp.float32),
              jax.ShapeDtypeStruct((E,), jnp.int32),
              jax.ShapeDtypeStruct((E,), jnp.int32)),
    mesh=_mesh,
    scratch_types=[
        [pltpu.VMEM((2, CD), jnp.int32)] * 3,
        [pltpu.VMEM((CD,), jnp.int32)] * 3,
        [pltpu.VMEM((CD,), jnp.int32)] * 3,
        [pltpu.VMEM((CD,), jnp.float32)] * 3,
        pltpu.VMEM((PT,), jnp.float32),
        pltpu.VMEM_SHARED((NP,), jnp.float32),
        [pltpu.SemaphoreType.DMA] * 3,
        [pltpu.SemaphoreType.DMA] * 3,
        [pltpu.SemaphoreType.DMA] * 3,
        [pltpu.SemaphoreType.DMA] * 3,
        [pltpu.SemaphoreType.DMA] * 3,
    ],
    compiler_params=pltpu.CompilerParams(needs_layout_passes=False),
)
def _deg_kernel(ei_hbm, ew_hbm, out_hbm, src_hbm, dst_hbm,
                exy, sflat, dflat, val_v, zbuf, acc,
                lsems_i, lsems_v, ssems, wsems_s, wsems_d):
    c = lax.axis_index("c")
    s = lax.axis_index("s")
    wid = s * NC + c

    def zb(i, _):
        zbuf[pl.ds(i * L, L)] = jnp.zeros((L,), jnp.float32)
        return 0
    lax.fori_loop(0, PT // L, zb, 0)
    pltpu.sync_copy(zbuf, acc.at[pl.ds(s * PT, PT)])
    plsc.subcore_barrier()

    def start_loads(j):
        b = j % 3
        off = (j * NW + wid) * CD
        di = pltpu.async_copy(ei_hbm.at[:, pl.ds(off, CD)], exy[b],
                              lsems_i[b])
        dv = pltpu.async_copy(ew_hbm.at[pl.ds(off, CD)], val_v[b],
                              lsems_v[b])
        return di, dv

    def deinterleave(b):
        def dcopy(i, _, b=b):
            for u in range(4):
                o = i * (4 * L) + u * L
                sflat[b][pl.ds(o, L)] = exy[b][0, pl.ds(o, L)]
                dflat[b][pl.ds(o, L)] = exy[b][1, pl.ds(o, L)]
            return 0
        lax.fori_loop(0, CD // (4 * L), dcopy, 0)

    loads = {0: start_loads(0)}
    pend = {}
    for j in range(RDN):
        b = j % 3
        di, dv = loads.pop(j)
        di.wait()
        dv.wait()
        if j + 1 < RDN:
            if j - 2 >= 0:
                for d in pend.pop(j - 2):
                    d.wait()
            loads[j + 1] = start_loads(j + 1)
        deinterleave(b)
        off = (j * NW + wid) * CD
        pend[j] = (
            pltpu.async_copy(val_v[b], acc.at[dflat[b]], ssems[b], add=True),
            pltpu.async_copy(sflat[b], src_hbm.at[pl.ds(off, CD)], wsems_s[b]),
            pltpu.async_copy(dflat[b], dst_hbm.at[pl.ds(off, CD)], wsems_d[b]),
        )
    for j in sorted(pend):
        for d in pend.pop(j):
            d.wait()

    @pl.when(wid < TAILD)
    def _tail():
        off = (RDN * NW + wid) * CD
        pltpu.sync_copy(ei_hbm.at[:, pl.ds(off, CD)], exy[0])
        pltpu.sync_copy(ew_hbm.at[pl.ds(off, CD)], val_v[0])
        deinterleave(0)
        pltpu.sync_copy(val_v[0], acc.at[dflat[0]], add=True)
        pltpu.sync_copy(sflat[0], src_hbm.at[pl.ds(off, CD)])
        pltpu.sync_copy(dflat[0], dst_hbm.at[pl.ds(off, CD)])

    plsc.subcore_barrier()
    pltpu.sync_copy(acc.at[pl.ds(s * PT, PT)],
                    out_hbm.at[pl.ds(c * NP + s * PT, PT)])


@functools.partial(
    pl.kernel,
    out_type=jax.ShapeDtypeStruct((NC * NP,), jnp.float32),
    mesh=_mesh,
    scratch_types=[
        pltpu.VMEM((NP,), jnp.float32),
        [pltpu.VMEM((C,), jnp.int32)] * 2,
        [pltpu.VMEM((C,), jnp.int32)] * 3,
        [pltpu.VMEM((C,), jnp.float32)] * 2,
        [pltpu.VMEM((C,), jnp.float32)] * 3,
        pltpu.VMEM_SHARED((NP,), jnp.float32),
        [pltpu.SemaphoreType.DMA] * 2,
        [pltpu.SemaphoreType.DMA] * 3,
        [pltpu.SemaphoreType.DMA] * 2,
        [pltpu.SemaphoreType.DMA] * 3,
        pltpu.SemaphoreType.DMA,
    ],
    compiler_params=pltpu.CompilerParams(needs_layout_passes=False),
)
def _edge_kernel(src_hbm, dst_hbm, ew_hbm, tbl_hbm, out_hbm,
                 tbl_v, sidx, didx, w_v, prod, acc,
                 sem_s, sem_d, sem_w, sem_sc, sem_t):
    c = lax.axis_index("c")
    s = lax.axis_index("s")
    wid = s * NC + c
    tload = pltpu.async_copy(tbl_hbm, tbl_v, sem_t)

    def zb(i, _):
        prod[0][pl.ds(i * L, L)] = jnp.zeros((L,), jnp.float32)
        return 0
    lax.fori_loop(0, C // L, zb, 0)
    for r in range(PT // C):
        pltpu.sync_copy(prod[0], acc.at[pl.ds(s * PT + r * C, C)])
    rem = PT % C
    if rem:
        pltpu.sync_copy(prod[0].at[pl.ds(0, rem)],
                        acc.at[pl.ds(s * PT + (PT // C) * C, rem)])
    tload.wait()
    plsc.subcore_barrier()
    base = wid * EW

    def start_loads(j):
        b2, b3 = j % 2, j % 3
        off = base + j * C
        ds_ = pltpu.async_copy(src_hbm.at[pl.ds(off, C)], sidx[b2],
                               sem_s[b2])
        dd = pltpu.async_copy(dst_hbm.at[pl.ds(off, C)], didx[b3],
                              sem_d[b3])
        dw = pltpu.async_copy(ew_hbm.at[pl.ds(off, C)], w_v[b2],
                              sem_w[b2])
        return ds_, dd, dw

    loads = {0: start_loads(0)}
    scats = {}
    for j in range(NCH):
        b2, b3 = j % 2, j % 3
        for d in loads.pop(j):
            d.wait()
        if j + 1 < NCH:
            if j - 2 >= 0:
                scats.pop(j - 2).wait()
            loads[j + 1] = start_loads(j + 1)

        def inner(i, _, b2=b2, b3=b3):
            for u in range(5):
                o = i * (5 * L) + u * L
                s16 = sidx[b2][pl.ds(o, L)]
                g16 = plsc.load_gather(tbl_v, [s16])
                prod[b3][pl.ds(o, L)] = g16 * w_v[b2][pl.ds(o, L)]
            return 0

        lax.fori_loop(0, C // (5 * L), inner, 0)
        scats[j] = pltpu.async_copy(prod[b3], acc.at[didx[b3]],
                                    sem_sc[b3], add=True)
    for j in sorted(scats):
        scats.pop(j).wait()
    plsc.subcore_barrier()
    pltpu.sync_copy(acc.at[pl.ds(s * PT, PT)],
                    out_hbm.at[pl.ds(c * NP + s * PT, PT)])


def _dinv_body(degp_ref, x_ref, dinv_ref, p_ref):
    deg = degp_ref[0] + degp_ref[1] + 1.0
    dinv = lax.rsqrt(deg)
    dinv_ref[...] = dinv
    p_ref[...] = dinv * x_ref[...]


_dinv_call = pl.pallas_call(
    _dinv_body,
    out_shape=(jax.ShapeDtypeStruct((RND, 128), jnp.float32),
               jax.ShapeDtypeStruct((RND, 128), jnp.float32)),
)


def _mlp_body(s1p_ref, dinv_ref, p_ref, w1_ref, b1_ref, w2_ref, q_ref):
    dinv = dinv_ref[...]
    a = dinv * (s1p_ref[0] + s1p_ref[1] + p_ref[...])
    t = jnp.zeros_like(a)
    for h in range(H):
        t = t + jnp.maximum(a * w1_ref[0, h] + b1_ref[0, h], 0.0) * w2_ref[0, h]
    q_ref[...] = dinv * t


_mlp_call = pl.pallas_call(
    _mlp_body,
    out_shape=jax.ShapeDtypeStruct((RND, 128), jnp.float32),
)


def _final_body(s2p_ref, dinv_ref, q_ref, batch_ref, b2_ref, out_ref):
    cv = dinv_ref[...] * (s2p_ref[0] + s2p_ref[1] + q_ref[...]) + b2_ref[0, 0]
    b = batch_ref[...]
    sums, cnts = [], []
    for g in range(G):
        m = b == g
        sums.append(jnp.sum(jnp.where(m, cv, 0.0)))
        cnts.append(jnp.sum(jnp.where(m, 1.0, 0.0)))
    out_ref[0, :] = jnp.stack(sums) / jnp.maximum(jnp.stack(cnts), 1.0)


_final_call = pl.pallas_call(
    _final_body,
    out_shape=jax.ShapeDtypeStruct((1, G), jnp.float32),
)


def kernel(x, edge_index, edge_weight, batch, W1, b1, W2, b2):
    x_p = jnp.pad(x, (0, NP - N)).reshape(RND, 128)
    batch_p = jnp.pad(batch, (0, NP - N), constant_values=G).reshape(RND, 128)
    w1 = W1.reshape(1, H)
    b1r = b1.reshape(1, H)
    w2 = W2.reshape(1, H)
    b2r = b2.reshape(1, 1)

    degp, src, dst = _deg_kernel(edge_index, edge_weight)
    degp = degp.reshape(NC, RND, 128)
    dinv2, p2 = _dinv_call(degp, x_p)
    s1p = _edge_kernel(src, dst, edge_weight,
                       p2.reshape(NP)).reshape(NC, RND, 128)
    q2 = _mlp_call(s1p, dinv2, p2, w1, b1r, w2)
    s2p = _edge_kernel(src, dst, edge_weight,
                       q2.reshape(NP)).reshape(NC, RND, 128)
    out2 = _final_call(s2p, dinv2, q2, batch_p, b2r)
    return out2.reshape(G)
